# baseline (speedup 1.0000x reference)
import functools

import jax
import jax.numpy as jnp
from jax import lax
from jax.experimental import pallas as pl
from jax.experimental.pallas import tpu as pltpu
from jax.experimental.pallas import tpu_sc as plsc

B = 16384
D = 64
NU = 1000000
NC_TAB = 100000

_info = plsc.get_sparse_core_info()
_NCORE, _NSUB = _info.num_cores, _info.num_subcores
_NW = _NCORE * _NSUB
_BPW = B // _NW
_FP = 16

_U_LAST = (NU - 1) // 128
_UGRP = 8
_GSEG = 4


def _sread(ref, i):
    return ref[pl.ds(i, 16)][0]


def _stage_segments(r_v, seg_s):
    seg_s[0] = 0

    def body(p, ns):
        new = lax.ne(lax.shift_right_logical(_sread(r_v, _FP + p), 7),
                     lax.shift_right_logical(_sread(r_v, _FP + p - 1), 7))

        def write(n):
            seg_s[n] = p
            return n + 1

        return lax.cond(new, write, lambda n: n, ns)

    ns = lax.fori_loop(1, _BPW, body, 1)
    seg_s[ns] = _BPW
    return ns


def _user_kernel(skeys, user_raw, user_tt, user_tail, emb_out,
                 seg_s, uraw_v, key_v, r_v, i_v, slot_v, col_v, stage,
                 buf, slabsem, wsem):
    wid = lax.axis_index("s") * _NCORE + lax.axis_index("c")
    base = wid * _BPW
    lanes = lax.iota(jnp.int32, 16)

    pltpu.sync_copy(user_raw, uraw_v)
    pltpu.sync_copy(skeys.at[pl.ds(base, _BPW)], key_v.at[pl.ds(_FP, _BPW)])
    pltpu.sync_copy(user_tail, buf.at[_UGRP])

    def decode(k, c):
        kv = key_v[pl.ds(_FP + k * 16, 16)]
        iv = lax.bitwise_and(kv, (1 << 14) - 1)
        rv = plsc.load_gather(uraw_v, [iv])
        i_v[pl.ds(k * 16, 16)] = iv
        r_v[pl.ds(_FP + k * 16, 16)] = rv
        return c

    lax.fori_loop(0, _BPW // 16, decode, 0)
    first = r_v[pl.ds(_FP, 16)]
    r_v[pl.ds(0, 16)] = jnp.full((16,), 1, jnp.int32) * first[0]

    ns = _stage_segments(r_v, seg_s)

    lastv = jnp.full((16,), _U_LAST, jnp.int32)

    def slots(k, segc):
        v = r_v[pl.ds(_FP + k * 16, 16)]
        prev = r_v[pl.ds(_FP + k * 16 - 1, 16)]
        vs = lax.shift_right_logical(v, 7)
        b = (vs != lax.shift_right_logical(prev, 7)).astype(jnp.int32)
        segs = plsc.cumsum(b) + segc
        is_last = vs == lastv
        slot = jnp.where(is_last, jnp.int32(_UGRP),
                         lax.bitwise_and(segs, _UGRP - 1))
        col = jnp.where(is_last, v - (NU - 128), lax.bitwise_and(v, 127))
        slot_v[pl.ds(k * 16, 16)] = slot
        col_v[pl.ds(k * 16, 16)] = col
        return segs[15]

    lax.fori_loop(0, _BPW // 16, slots, jnp.int32(0))

    ngrp = (ns + _GSEG - 1) // _GSEG

    def gseg(g, j):
        s = g * _GSEG + j
        valid = s < ns
        st = seg_s[jnp.minimum(s, ns)]
        slab = lax.shift_right_logical(_sread(r_v, _FP + st), 7)
        return jnp.logical_and(valid, lax.ne(slab, _U_LAST)), slab, s

    def gfire(g):
        for j in range(_GSEG):
            fire, slab, s = gseg(g, j)

            @pl.when(fire)
            def _():
                off = pl.multiple_of(slab * 128, 128)
                pltpu.async_copy(user_tt.at[:, pl.ds(off, 128)],
                                 buf.at[lax.bitwise_and(s, _UGRP - 1)],
                                 slabsem)

    gfire(jnp.int32(0))

    def group(g, carry):
        nf = jnp.int32(0)
        for j in range(_GSEG):
            fire, _, _ = gseg(g, j)
            nf = nf + fire.astype(jnp.int32)

        def drain(i, c):
            pltpu.make_async_copy(user_tt.at[:, pl.ds(0, 128)],
                                  buf.at[0], slabsem).wait()
            return c

        lax.fori_loop(0, nf, drain, 0)
        gfire(g + 1)

        pstart = seg_s[jnp.minimum(g * _GSEG, ns)]
        pend = seg_s[jnp.minimum((g + 1) * _GSEG, ns)]

        def pblock(pb, c):
            pos = pb * 16 + lanes
            m = jnp.logical_and(pos >= pstart, pos < pend)
            sl = slot_v[pl.ds(pb * 16, 16)]
            co = col_v[pl.ds(pb * 16, 16)]
            wbase = pos * D
            for d in range(D):
                dv = jnp.full((16,), d, jnp.int32)
                val = plsc.load_gather(buf, [sl, dv, co])
                plsc.store_scatter(stage, [wbase + d], val, mask=m)
            return c

        lax.fori_loop(lax.shift_right_logical(pstart, 4),
                      lax.shift_right_logical(pend + 15, 4), pblock, 0)
        return carry

    lax.fori_loop(0, ngrp, group, 0)

    def emit(p, c):
        i = _sread(i_v, p)
        pltpu.async_copy(
            stage.at[pl.ds(pl.multiple_of(p * D, D), D)],
            emb_out.at[pl.ds(pl.multiple_of(i * D, D), D)], wsem)
        return c

    lax.fori_loop(0, _BPW, emit, 0)
    pltpu.make_async_copy(stage, emb_out.at[pl.ds(0, _BPW * D)],
                          wsem).wait()


def _content_kernel(content_idx, content_table, u_emb, out_hbm,
                    cidx_v, crows, urows, out_v, csem):
    wid = lax.axis_index("s") * _NCORE + lax.axis_index("c")
    base = wid * _BPW
    lanes = lax.iota(jnp.int32, 16)

    pltpu.sync_copy(content_idx.at[pl.ds(wid * 4, 4)], cidx_v)
    pltpu.sync_copy(u_emb.at[pl.ds(base * D, _BPW * D)], urows)

    copies = []
    for j in range(4):
        copies.append(pltpu.async_copy(
            content_table.at[cidx_v.at[j]],
            crows.at[pl.ds(j * 128, 128)], csem))
    for c in copies:
        c.wait()

    def dblk(h, c):
        prow = lanes + h * 16
        rows = prow * D
        acc0 = jnp.zeros((16,), jnp.float32)
        acc1 = jnp.zeros((16,), jnp.float32)
        for d in range(0, D, 2):
            d0 = jnp.full((16,), d, jnp.int32)
            d1 = jnp.full((16,), d + 1, jnp.int32)
            acc0 += (plsc.load_gather(urows, [rows + d])
                     * plsc.load_gather(crows, [prow, d0]))
            acc1 += (plsc.load_gather(urows, [rows + (d + 1)])
                     * plsc.load_gather(crows, [prow, d1]))
        out_v[pl.ds(h * 16, 16)] = acc0 + acc1
        return c

    lax.fori_loop(0, _BPW // 16, dblk, 0)

    pltpu.sync_copy(out_v, out_hbm.at[pl.ds(base, _BPW)])


@jax.jit
def _run(skeys, user, content_idx2d, user_tt, content_table, user_tail):
    mesh = plsc.VectorSubcoreMesh(core_axis_name="c", subcore_axis_name="s")

    k1 = functools.partial(
        pl.kernel, mesh=mesh,
        out_type=jax.ShapeDtypeStruct((B * D,), jnp.float32),
        compiler_params=pltpu.CompilerParams(needs_layout_passes=False),
        scratch_types=[
            pltpu.SMEM((_BPW + 1,), jnp.int32),
            pltpu.VMEM((B,), jnp.int32),
            pltpu.VMEM((_FP + _BPW + 16,), jnp.int32),
            pltpu.VMEM((_FP + _BPW + 16,), jnp.int32),
            pltpu.VMEM((_BPW + 16,), jnp.int32),
            pltpu.VMEM((_BPW + 16,), jnp.int32),
            pltpu.VMEM((_BPW + 16,), jnp.int32),
            pltpu.VMEM((_BPW * D,), jnp.float32),
            pltpu.VMEM((_UGRP + 1, D, 128), jnp.float32),
            pltpu.SemaphoreType.DMA,
            pltpu.SemaphoreType.DMA,
        ],
    )(_user_kernel)
    u_emb = k1(skeys, user, user_tt, user_tail)

    k2 = functools.partial(
        pl.kernel, mesh=mesh,
        out_type=jax.ShapeDtypeStruct((B,), jnp.float32),
        compiler_params=pltpu.CompilerParams(
            needs_layout_passes=False, use_tc_tiling_on_sc=False),
        scratch_types=[
            pltpu.VMEM((4, 128), jnp.int32),
            pltpu.VMEM((_BPW, D), jnp.float32),
            pltpu.VMEM((_BPW * D,), jnp.float32),
            pltpu.VMEM((_BPW,), jnp.float32),
            pltpu.SemaphoreType.DMA,
        ],
    )(_content_kernel)
    return k2(content_idx2d, content_table, u_emb)


def kernel(user, content, user_table, content_table):
    keys = lax.bitwise_or(
        lax.shift_left(lax.shift_right_logical(user, 7), 14),
        jnp.arange(B, dtype=jnp.int32))
    skeys = lax.sort(keys, is_stable=False)
    out = _run(skeys, user, content.reshape(B // 128, 128),
               user_table.T, content_table,
               user_table[NU - 128:, :].T)
    return out.reshape(B, 1)


# speedup vs baseline: 1.0048x; 1.0048x over previous
import functools

import jax
import jax.numpy as jnp
from jax import lax
from jax.experimental import pallas as pl
from jax.experimental.pallas import tpu as pltpu
from jax.experimental.pallas import tpu_sc as plsc

B = 16384
D = 64
NU = 1000000
NC_TAB = 100000

_info = plsc.get_sparse_core_info()
_NCORE, _NSUB = _info.num_cores, _info.num_subcores
_NW = _NCORE * _NSUB
_BPW = B // _NW
_FP = 16

_U_LAST = (NU - 1) // 128
_UGRP = 8
_GSEG = 4


def _sread(ref, i):
    return ref[pl.ds(i, 16)][0]


def _stage_segments(r_v, seg_s):
    seg_s[0] = 0

    def body(p, ns):
        new = lax.ne(lax.shift_right_logical(_sread(r_v, _FP + p), 7),
                     lax.shift_right_logical(_sread(r_v, _FP + p - 1), 7))

        def write(n):
            seg_s[n] = p
            return n + 1

        return lax.cond(new, write, lambda n: n, ns)

    ns = lax.fori_loop(1, _BPW, body, 1)
    seg_s[ns] = _BPW
    return ns


def _user_kernel(skeys, user_raw, user_tt, user_tail, emb_out,
                 seg_s, uraw_v, key_v, r_v, i_v, slot_v, col_v, stage,
                 buf, slabsem, wsem):
    wid = lax.axis_index("s") * _NCORE + lax.axis_index("c")
    base = wid * _BPW
    lanes = lax.iota(jnp.int32, 16)

    pltpu.sync_copy(user_raw, uraw_v)
    pltpu.sync_copy(skeys.at[pl.ds(base, _BPW)], key_v.at[pl.ds(_FP, _BPW)])
    pltpu.sync_copy(user_tail, buf.at[_UGRP])

    def decode(k, c):
        kv = key_v[pl.ds(_FP + k * 16, 16)]
        iv = lax.bitwise_and(kv, (1 << 14) - 1)
        rv = plsc.load_gather(uraw_v, [iv])
        i_v[pl.ds(k * 16, 16)] = iv
        r_v[pl.ds(_FP + k * 16, 16)] = rv
        return c

    lax.fori_loop(0, _BPW // 16, decode, 0)
    first = r_v[pl.ds(_FP, 16)]
    r_v[pl.ds(0, 16)] = jnp.full((16,), 1, jnp.int32) * first[0]

    ns = _stage_segments(r_v, seg_s)

    lastv = jnp.full((16,), _U_LAST, jnp.int32)

    def slots(k, segc):
        v = r_v[pl.ds(_FP + k * 16, 16)]
        prev = r_v[pl.ds(_FP + k * 16 - 1, 16)]
        vs = lax.shift_right_logical(v, 7)
        b = (vs != lax.shift_right_logical(prev, 7)).astype(jnp.int32)
        segs = plsc.cumsum(b) + segc
        is_last = vs == lastv
        slot = jnp.where(is_last, jnp.int32(_UGRP),
                         lax.bitwise_and(segs, _UGRP - 1))
        col = jnp.where(is_last, v - (NU - 128), lax.bitwise_and(v, 127))
        slot_v[pl.ds(k * 16, 16)] = slot
        col_v[pl.ds(k * 16, 16)] = col
        return segs[15]

    lax.fori_loop(0, _BPW // 16, slots, jnp.int32(0))

    ngrp = (ns + _GSEG - 1) // _GSEG

    def gseg(g, j):
        s = g * _GSEG + j
        valid = s < ns
        st = seg_s[jnp.minimum(s, ns)]
        slab = lax.shift_right_logical(_sread(r_v, _FP + st), 7)
        return jnp.logical_and(valid, lax.ne(slab, _U_LAST)), slab, s

    def gfire(g):
        for j in range(_GSEG):
            fire, slab, s = gseg(g, j)

            @pl.when(fire)
            def _():
                off = pl.multiple_of(slab * 128, 128)
                pltpu.async_copy(user_tt.at[:, pl.ds(off, 128)],
                                 buf.at[lax.bitwise_and(s, _UGRP - 1)],
                                 slabsem)

    gfire(jnp.int32(0))

    def group(g, carry):
        nf = jnp.int32(0)
        for j in range(_GSEG):
            fire, _, _ = gseg(g, j)
            nf = nf + fire.astype(jnp.int32)

        def drain(i, c):
            pltpu.make_async_copy(user_tt.at[:, pl.ds(0, 128)],
                                  buf.at[0], slabsem).wait()
            return c

        lax.fori_loop(0, nf, drain, 0)
        gfire(g + 1)

        pstart = seg_s[jnp.minimum(g * _GSEG, ns)]
        pend = seg_s[jnp.minimum((g + 1) * _GSEG, ns)]

        def pblock(pb, c):
            pos = pb * 16 + lanes
            m = jnp.logical_and(pos >= pstart, pos < pend)
            sl = slot_v[pl.ds(pb * 16, 16)]
            co = col_v[pl.ds(pb * 16, 16)]
            wbase = pos * D
            for d in range(D):
                dv = jnp.full((16,), d, jnp.int32)
                val = plsc.load_gather(buf, [sl, dv, co])
                plsc.store_scatter(stage, [wbase + d], val, mask=m)
            return c

        lax.fori_loop(lax.shift_right_logical(pstart, 4),
                      lax.shift_right_logical(pend + 15, 4), pblock, 0)
        return carry

    lax.fori_loop(0, ngrp, group, 0)

    def emit(p, c):
        i = _sread(i_v, p)
        pltpu.async_copy(
            stage.at[pl.ds(pl.multiple_of(p * D, D), D)],
            emb_out.at[pl.ds(pl.multiple_of(i * D, D), D)], wsem)
        return c

    lax.fori_loop(0, _BPW, emit, 0)
    pltpu.make_async_copy(stage, emb_out.at[pl.ds(0, _BPW * D)],
                          wsem).wait()


def _content_kernel(content_idx, content_table, u_emb, out_hbm,
                    cidx_v, crows, urows, out_v, csem, usem):
    wid = lax.axis_index("s") * _NCORE + lax.axis_index("c")
    base = wid * _BPW
    lanes = lax.iota(jnp.int32, 16)

    pltpu.sync_copy(content_idx.at[pl.ds(wid * 4, 4)], cidx_v)
    ucp = pltpu.async_copy(u_emb.at[pl.ds(base * D, _BPW * D)], urows,
                           usem)

    copies = []
    for j in range(4):
        copies.append(pltpu.async_copy(
            content_table.at[cidx_v.at[j]],
            crows.at[pl.ds(j * 128, 128)], csem))
    for c in copies:
        c.wait()
    ucp.wait()

    def dblk(h, c):
        prow = lanes + h * 16
        rows = prow * D
        acc0 = jnp.zeros((16,), jnp.float32)
        acc1 = jnp.zeros((16,), jnp.float32)
        for d in range(0, D, 2):
            d0 = jnp.full((16,), d, jnp.int32)
            d1 = jnp.full((16,), d + 1, jnp.int32)
            acc0 += (plsc.load_gather(urows, [rows + d])
                     * plsc.load_gather(crows, [prow, d0]))
            acc1 += (plsc.load_gather(urows, [rows + (d + 1)])
                     * plsc.load_gather(crows, [prow, d1]))
        out_v[pl.ds(h * 16, 16)] = acc0 + acc1
        return c

    lax.fori_loop(0, _BPW // 16, dblk, 0)

    pltpu.sync_copy(out_v, out_hbm.at[pl.ds(base, _BPW)])


@jax.jit
def _run(skeys, user, content_idx2d, user_tt, content_table, user_tail):
    mesh = plsc.VectorSubcoreMesh(core_axis_name="c", subcore_axis_name="s")

    k1 = functools.partial(
        pl.kernel, mesh=mesh,
        out_type=jax.ShapeDtypeStruct((B * D,), jnp.float32),
        compiler_params=pltpu.CompilerParams(needs_layout_passes=False),
        scratch_types=[
            pltpu.SMEM((_BPW + 1,), jnp.int32),
            pltpu.VMEM((B,), jnp.int32),
            pltpu.VMEM((_FP + _BPW + 16,), jnp.int32),
            pltpu.VMEM((_FP + _BPW + 16,), jnp.int32),
            pltpu.VMEM((_BPW + 16,), jnp.int32),
            pltpu.VMEM((_BPW + 16,), jnp.int32),
            pltpu.VMEM((_BPW + 16,), jnp.int32),
            pltpu.VMEM((_BPW * D,), jnp.float32),
            pltpu.VMEM((_UGRP + 1, D, 128), jnp.float32),
            pltpu.SemaphoreType.DMA,
            pltpu.SemaphoreType.DMA,
        ],
    )(_user_kernel)
    u_emb = k1(skeys, user, user_tt, user_tail)

    k2 = functools.partial(
        pl.kernel, mesh=mesh,
        out_type=jax.ShapeDtypeStruct((B,), jnp.float32),
        compiler_params=pltpu.CompilerParams(
            needs_layout_passes=False, use_tc_tiling_on_sc=False),
        scratch_types=[
            pltpu.VMEM((4, 128), jnp.int32),
            pltpu.VMEM((_BPW, D), jnp.float32),
            pltpu.VMEM((_BPW * D,), jnp.float32),
            pltpu.VMEM((_BPW,), jnp.float32),
            pltpu.SemaphoreType.DMA,
            pltpu.SemaphoreType.DMA,
        ],
    )(_content_kernel)
    return k2(content_idx2d, content_table, u_emb)


def kernel(user, content, user_table, content_table):
    keys = lax.bitwise_or(
        lax.shift_left(lax.shift_right_logical(user, 7), 14),
        jnp.arange(B, dtype=jnp.int32))
    skeys = lax.sort(keys, is_stable=False)
    out = _run(skeys, user, content.reshape(B // 128, 128),
               user_table.T, content_table,
               user_table[NU - 128:, :].T)
    return out.reshape(B, 1)


# speedup vs baseline: 1.0319x; 1.0270x over previous
import functools

import jax
import jax.numpy as jnp
from jax import lax
from jax.experimental import pallas as pl
from jax.experimental.pallas import tpu as pltpu
from jax.experimental.pallas import tpu_sc as plsc

B = 16384
D = 64
NU = 1000000
NC_TAB = 100000

_info = plsc.get_sparse_core_info()
_NCORE, _NSUB = _info.num_cores, _info.num_subcores
_NW = _NCORE * _NSUB
_BPW = B // _NW
_FP = 16

_U_LAST = (NU - 1) // 128
_UGRP = 8
_GSEG = 4


def _sread(ref, i):
    return ref[pl.ds(i, 16)][0]


def _stage_segments(r_v, seg_s):
    seg_s[0] = 0

    def body(p, ns):
        new = lax.ne(lax.shift_right_logical(_sread(r_v, _FP + p), 7),
                     lax.shift_right_logical(_sread(r_v, _FP + p - 1), 7))

        def write(n):
            seg_s[n] = p
            return n + 1

        return lax.cond(new, write, lambda n: n, ns)

    ns = lax.fori_loop(1, _BPW, body, 1)
    seg_s[ns] = _BPW
    return ns


def _user_kernel(skeys, user_raw, user_tt, user_tail, emb_out,
                 seg_s, uraw_v, key_v, r_v, i_v, slot_v, col_v, stage,
                 buf, slabsem, wsem):
    wid = lax.axis_index("s") * _NCORE + lax.axis_index("c")
    base = wid * _BPW
    lanes = lax.iota(jnp.int32, 16)

    pltpu.sync_copy(user_raw, uraw_v)
    pltpu.sync_copy(skeys.at[pl.ds(base, _BPW)], key_v.at[pl.ds(_FP, _BPW)])
    pltpu.sync_copy(user_tail, buf.at[_UGRP])

    def decode(k, c):
        kv = key_v[pl.ds(_FP + k * 16, 16)]
        iv = lax.bitwise_and(kv, (1 << 14) - 1)
        rv = plsc.load_gather(uraw_v, [iv])
        i_v[pl.ds(k * 16, 16)] = iv
        r_v[pl.ds(_FP + k * 16, 16)] = rv
        return c

    lax.fori_loop(0, _BPW // 16, decode, 0)
    first = r_v[pl.ds(_FP, 16)]
    r_v[pl.ds(0, 16)] = jnp.full((16,), 1, jnp.int32) * first[0]

    ns = _stage_segments(r_v, seg_s)

    lastv = jnp.full((16,), _U_LAST, jnp.int32)

    def slots(k, segc):
        v = r_v[pl.ds(_FP + k * 16, 16)]
        prev = r_v[pl.ds(_FP + k * 16 - 1, 16)]
        vs = lax.shift_right_logical(v, 7)
        b = (vs != lax.shift_right_logical(prev, 7)).astype(jnp.int32)
        segs = plsc.cumsum(b) + segc
        is_last = vs == lastv
        slot = jnp.where(is_last, jnp.int32(_UGRP),
                         lax.bitwise_and(segs, _UGRP - 1))
        col = jnp.where(is_last, v - (NU - 128), lax.bitwise_and(v, 127))
        slot_v[pl.ds(k * 16, 16)] = slot
        col_v[pl.ds(k * 16, 16)] = col
        return segs[15]

    lax.fori_loop(0, _BPW // 16, slots, jnp.int32(0))

    ngrp = (ns + _GSEG - 1) // _GSEG

    def gseg(g, j):
        s = g * _GSEG + j
        valid = s < ns
        st = seg_s[jnp.minimum(s, ns)]
        slab = lax.shift_right_logical(_sread(r_v, _FP + st), 7)
        return jnp.logical_and(valid, lax.ne(slab, _U_LAST)), slab, s

    def gfire(g):
        for j in range(_GSEG):
            fire, slab, s = gseg(g, j)

            @pl.when(fire)
            def _():
                off = pl.multiple_of(slab * 128, 128)
                pltpu.async_copy(user_tt.at[:, pl.ds(off, 128)],
                                 buf.at[lax.bitwise_and(s, _UGRP - 1)],
                                 slabsem)

    gfire(jnp.int32(0))

    def group(g, carry):
        nf = jnp.int32(0)
        for j in range(_GSEG):
            fire, _, _ = gseg(g, j)
            nf = nf + fire.astype(jnp.int32)

        def drain(i, c):
            pltpu.make_async_copy(user_tt.at[:, pl.ds(0, 128)],
                                  buf.at[0], slabsem).wait()
            return c

        lax.fori_loop(0, nf, drain, 0)
        gfire(g + 1)

        pstart = seg_s[jnp.minimum(g * _GSEG, ns)]
        pend = seg_s[jnp.minimum((g + 1) * _GSEG, ns)]

        def pblock(pb, c):
            pos = pb * 16 + lanes
            m = jnp.logical_and(pos >= pstart, pos < pend)
            sl = slot_v[pl.ds(pb * 16, 16)]
            co = col_v[pl.ds(pb * 16, 16)]
            wbase = pos * D
            for d in range(D):
                dv = jnp.full((16,), d, jnp.int32)
                val = plsc.load_gather(buf, [sl, dv, co])
                plsc.store_scatter(stage, [wbase + d], val, mask=m)
            return c

        lax.fori_loop(lax.shift_right_logical(pstart, 4),
                      lax.shift_right_logical(pend + 15, 4), pblock, 0)

        def emit(p, c):
            i = _sread(i_v, p)
            pltpu.async_copy(
                stage.at[pl.ds(pl.multiple_of(p * D, D), D)],
                emb_out.at[pl.ds(pl.multiple_of(i * D, D), D)], wsem)
            return c

        lax.fori_loop(pstart, pend, emit, 0)
        return carry

    lax.fori_loop(0, ngrp, group, 0)
    pltpu.make_async_copy(stage, emb_out.at[pl.ds(0, _BPW * D)],
                          wsem).wait()


def _content_kernel(content_idx, content_table, u_emb, out_hbm,
                    cidx_v, crows, urows, out_v, csem, usem):
    wid = lax.axis_index("s") * _NCORE + lax.axis_index("c")
    base = wid * _BPW
    lanes = lax.iota(jnp.int32, 16)

    pltpu.sync_copy(content_idx.at[pl.ds(wid * 4, 4)], cidx_v)
    ucp = pltpu.async_copy(u_emb.at[pl.ds(base * D, _BPW * D)], urows,
                           usem)

    copies = []
    for j in range(4):
        copies.append(pltpu.async_copy(
            content_table.at[cidx_v.at[j]],
            crows.at[pl.ds(j * 128, 128)], csem))
    for c in copies:
        c.wait()
    ucp.wait()

    def dblk(h, c):
        prow = lanes + h * 16
        rows = prow * D
        acc0 = jnp.zeros((16,), jnp.float32)
        acc1 = jnp.zeros((16,), jnp.float32)
        for d in range(0, D, 2):
            d0 = jnp.full((16,), d, jnp.int32)
            d1 = jnp.full((16,), d + 1, jnp.int32)
            acc0 += (plsc.load_gather(urows, [rows + d])
                     * plsc.load_gather(crows, [prow, d0]))
            acc1 += (plsc.load_gather(urows, [rows + (d + 1)])
                     * plsc.load_gather(crows, [prow, d1]))
        out_v[pl.ds(h * 16, 16)] = acc0 + acc1
        return c

    lax.fori_loop(0, _BPW // 16, dblk, 0)

    pltpu.sync_copy(out_v, out_hbm.at[pl.ds(base, _BPW)])


@jax.jit
def _run(skeys, user, content_idx2d, user_tt, content_table, user_tail):
    mesh = plsc.VectorSubcoreMesh(core_axis_name="c", subcore_axis_name="s")

    k1 = functools.partial(
        pl.kernel, mesh=mesh,
        out_type=jax.ShapeDtypeStruct((B * D,), jnp.float32),
        compiler_params=pltpu.CompilerParams(needs_layout_passes=False),
        scratch_types=[
            pltpu.SMEM((_BPW + 1,), jnp.int32),
            pltpu.VMEM((B,), jnp.int32),
            pltpu.VMEM((_FP + _BPW + 16,), jnp.int32),
            pltpu.VMEM((_FP + _BPW + 16,), jnp.int32),
            pltpu.VMEM((_BPW + 16,), jnp.int32),
            pltpu.VMEM((_BPW + 16,), jnp.int32),
            pltpu.VMEM((_BPW + 16,), jnp.int32),
            pltpu.VMEM((_BPW * D,), jnp.float32),
            pltpu.VMEM((_UGRP + 1, D, 128), jnp.float32),
            pltpu.SemaphoreType.DMA,
            pltpu.SemaphoreType.DMA,
        ],
    )(_user_kernel)
    u_emb = k1(skeys, user, user_tt, user_tail)

    k2 = functools.partial(
        pl.kernel, mesh=mesh,
        out_type=jax.ShapeDtypeStruct((B,), jnp.float32),
        compiler_params=pltpu.CompilerParams(
            needs_layout_passes=False, use_tc_tiling_on_sc=False),
        scratch_types=[
            pltpu.VMEM((4, 128), jnp.int32),
            pltpu.VMEM((_BPW, D), jnp.float32),
            pltpu.VMEM((_BPW * D,), jnp.float32),
            pltpu.VMEM((_BPW,), jnp.float32),
            pltpu.SemaphoreType.DMA,
            pltpu.SemaphoreType.DMA,
        ],
    )(_content_kernel)
    return k2(content_idx2d, content_table, u_emb)


def kernel(user, content, user_table, content_table):
    keys = lax.bitwise_or(
        lax.shift_left(lax.shift_right_logical(user, 7), 14),
        jnp.arange(B, dtype=jnp.int32))
    skeys = lax.sort(keys, is_stable=False)
    out = _run(skeys, user, content.reshape(B // 128, 128),
               user_table.T, content_table,
               user_table[NU - 128:, :].T)
    return out.reshape(B, 1)
